# baseline (speedup 1.0000x reference)
import jax
import jax.numpy as jnp
from jax.experimental import pallas as pl
from jax.experimental.pallas import tpu as pltpu

_CHUNKS = (64, 32, 16, 8, 4, 4)


def _loss_kernel(x_hbm, o_ref, buf, sems):
    C, H, W = buf.shape

    starts = []
    s = 0
    for sz in _CHUNKS:
        starts.append(s)
        s += sz

    for i, (c0, sz) in enumerate(zip(starts, _CHUNKS)):
        pltpu.make_async_copy(
            x_hbm.at[0, pl.ds(c0, sz)],
            buf.at[pl.ds(c0, sz)],
            sems.at[i],
        ).start()

    def wait_chunk(i, c0, sz):
        pltpu.make_async_copy(
            x_hbm.at[0, pl.ds(c0, sz)],
            buf.at[pl.ds(c0, sz)],
            sems.at[i],
        ).wait()

    wait_chunk(0, starts[0], _CHUNKS[0])
    m = buf[0]
    row = jax.lax.broadcasted_iota(jnp.int32, (H, W), 0)
    col = jax.lax.broadcasted_iota(jnp.int32, (H, W), 1)
    lin = row * W + col
    mv = jnp.max(m)
    idx = jnp.min(jnp.where(m == mv, lin, jnp.int32(H * W)))
    x0 = (idx // W).astype(jnp.float32)
    y0 = (idx % W).astype(jnp.float32)

    krow = row.astype(jnp.float32)
    b_vec = (y0 - krow) ** 2

    def chan_weight(j):
        aj = x0 - jnp.float32(j)
        return b_vec + aj * aj

    acc = buf[0] * chan_weight(0)
    for j in range(1, _CHUNKS[0]):
        acc = acc + buf[j] * chan_weight(j)

    for i in range(1, len(_CHUNKS)):
        c0, sz = starts[i], _CHUNKS[i]
        wait_chunk(i, c0, sz)
        for j in range(c0, c0 + sz):
            acc = acc + buf[j] * chan_weight(j)

    o_ref[:] = jnp.sum(acc, axis=0, keepdims=True)


def kernel(tensor):
    B, C, H, W = tensor.shape
    out = pl.pallas_call(
        _loss_kernel,
        out_shape=jax.ShapeDtypeStruct((1, W), jnp.float32),
        in_specs=[pl.BlockSpec(memory_space=pl.ANY)],
        out_specs=pl.BlockSpec(memory_space=pltpu.VMEM),
        scratch_shapes=[
            pltpu.VMEM((C, H, W), jnp.float32),
            pltpu.SemaphoreType.DMA((len(_CHUNKS),)),
        ],
    )(tensor)
    return out[0]


# speedup vs baseline: 1.0069x; 1.0069x over previous
import jax
import jax.numpy as jnp
from jax.experimental import pallas as pl
from jax.experimental.pallas import tpu as pltpu

_CHUNKS = (64, 32, 16, 8, 8)


def _loss_kernel(x_hbm, o_ref, buf, sems):
    C, H, W = buf.shape

    starts = []
    s = 0
    for sz in _CHUNKS:
        starts.append(s)
        s += sz

    for i, (c0, sz) in enumerate(zip(starts, _CHUNKS)):
        pltpu.make_async_copy(
            x_hbm.at[0, pl.ds(c0, sz)],
            buf.at[pl.ds(c0, sz)],
            sems.at[i],
        ).start()

    def wait_chunk(i, c0, sz):
        pltpu.make_async_copy(
            x_hbm.at[0, pl.ds(c0, sz)],
            buf.at[pl.ds(c0, sz)],
            sems.at[i],
        ).wait()

    wait_chunk(0, starts[0], _CHUNKS[0])
    m = buf[0]
    row = jax.lax.broadcasted_iota(jnp.int32, (H, W), 0)
    col = jax.lax.broadcasted_iota(jnp.int32, (H, W), 1)
    lin = row * W + col
    mv = jnp.max(m)
    idx = jnp.min(jnp.where(m == mv, lin, jnp.int32(H * W)))
    x0 = (idx // W).astype(jnp.float32)
    y0 = (idx % W).astype(jnp.float32)

    krow = row.astype(jnp.float32)
    b_vec = (y0 - krow) ** 2

    def chan_weight(j):
        aj = x0 - jnp.float32(j)
        return b_vec + aj * aj

    acc = buf[0] * chan_weight(0)
    for j in range(1, _CHUNKS[0]):
        acc = acc + buf[j] * chan_weight(j)

    for i in range(1, len(_CHUNKS)):
        c0, sz = starts[i], _CHUNKS[i]
        wait_chunk(i, c0, sz)
        for j in range(c0, c0 + sz):
            acc = acc + buf[j] * chan_weight(j)

    o_ref[:] = jnp.sum(acc, axis=0, keepdims=True)


def kernel(tensor):
    B, C, H, W = tensor.shape
    out = pl.pallas_call(
        _loss_kernel,
        out_shape=jax.ShapeDtypeStruct((1, W), jnp.float32),
        in_specs=[pl.BlockSpec(memory_space=pl.ANY)],
        out_specs=pl.BlockSpec(memory_space=pltpu.VMEM),
        scratch_shapes=[
            pltpu.VMEM((C, H, W), jnp.float32),
            pltpu.SemaphoreType.DMA((len(_CHUNKS),)),
        ],
    )(tensor)
    return out[0]


# speedup vs baseline: 1.0297x; 1.0227x over previous
import jax
import jax.numpy as jnp
from jax.experimental import pallas as pl
from jax.experimental.pallas import tpu as pltpu

_CHUNKS = (8, 56, 32, 16, 8, 8)


def _loss_kernel(x_hbm, o_ref, buf, sems):
    C, H, W = buf.shape

    starts = []
    s = 0
    for sz in _CHUNKS:
        starts.append(s)
        s += sz

    for i, (c0, sz) in enumerate(zip(starts, _CHUNKS)):
        pltpu.make_async_copy(
            x_hbm.at[0, pl.ds(c0, sz)],
            buf.at[pl.ds(c0, sz)],
            sems.at[i],
        ).start()

    def wait_chunk(i, c0, sz):
        pltpu.make_async_copy(
            x_hbm.at[0, pl.ds(c0, sz)],
            buf.at[pl.ds(c0, sz)],
            sems.at[i],
        ).wait()

    wait_chunk(0, starts[0], _CHUNKS[0])
    m = buf[0]
    row = jax.lax.broadcasted_iota(jnp.int32, (H, W), 0)
    col = jax.lax.broadcasted_iota(jnp.int32, (H, W), 1)
    lin = row * W + col
    mv = jnp.max(m)
    idx = jnp.min(jnp.where(m == mv, lin, jnp.int32(H * W)))
    x0 = (idx // W).astype(jnp.float32)
    y0 = (idx % W).astype(jnp.float32)

    krow = row.astype(jnp.float32)
    b_vec = (y0 - krow) ** 2

    def chan_weight(j):
        aj = x0 - jnp.float32(j)
        return b_vec + aj * aj

    acc = buf[0] * chan_weight(0)
    for j in range(1, _CHUNKS[0]):
        acc = acc + buf[j] * chan_weight(j)

    for i in range(1, len(_CHUNKS)):
        c0, sz = starts[i], _CHUNKS[i]
        wait_chunk(i, c0, sz)
        for j in range(c0, c0 + sz):
            acc = acc + buf[j] * chan_weight(j)

    o_ref[:] = jnp.sum(acc, axis=0, keepdims=True)


def kernel(tensor):
    B, C, H, W = tensor.shape
    out = pl.pallas_call(
        _loss_kernel,
        out_shape=jax.ShapeDtypeStruct((1, W), jnp.float32),
        in_specs=[pl.BlockSpec(memory_space=pl.ANY)],
        out_specs=pl.BlockSpec(memory_space=pltpu.VMEM),
        scratch_shapes=[
            pltpu.VMEM((C, H, W), jnp.float32),
            pltpu.SemaphoreType.DMA((len(_CHUNKS),)),
        ],
    )(tensor)
    return out[0]
